# baseline (speedup 1.0000x reference)
import jax
import jax.numpy as jnp
from jax.experimental import pallas as pl
from jax.experimental.pallas import tpu as pltpu

_NBUF = 3
_SL = 6


def _dwt_l1_body(p5, t5, out_ref, pe, po, te, to, se, so, ue, uo):
    nbuf, sl, hh, w = pe.shape
    n_local = pl.num_programs(1)
    ci = pl.program_id(0)
    i = pl.program_id(1)
    base = ci * n_local

    def start(k_local, slot):
        blk = (base + k_local) * sl
        src_p = p5.at[pl.ds(blk, sl)]
        src_t = t5.at[pl.ds(blk, sl)]
        pltpu.make_async_copy(src_p.at[:, :, 0, :], pe.at[slot], se.at[slot]).start()
        pltpu.make_async_copy(src_p.at[:, :, 1, :], po.at[slot], so.at[slot]).start()
        pltpu.make_async_copy(src_t.at[:, :, 0, :], te.at[slot], ue.at[slot]).start()
        pltpu.make_async_copy(src_t.at[:, :, 1, :], to.at[slot], uo.at[slot]).start()

    @pl.when(i == 0)
    def _prologue():
        for k in range(_NBUF - 1):
            start(k, k)

    nxt = i + _NBUF - 1

    @pl.when(nxt < n_local)
    def _lookahead():
        start(nxt, jax.lax.rem(nxt, _NBUF))

    slot = jax.lax.rem(i, _NBUF)
    for buf, sem in ((pe, se), (po, so), (te, ue), (to, uo)):
        pltpu.make_async_copy(buf.at[slot], buf.at[slot], sem.at[slot]).wait()

    L = 128
    acc = jnp.zeros((hh // 8, 8, L), jnp.float32)
    for s in range(sl):
        for c0 in range(0, w, L):
            ee = (pe[slot, s, :, c0:c0 + L]
                  - te[slot, s, :, c0:c0 + L]).reshape(hh // 8, 8, L)
            eo = (po[slot, s, :, c0:c0 + L]
                  - to[slot, s, :, c0:c0 + L]).reshape(hh // 8, 8, L)
            av = jnp.abs(ee + eo)
            aw = jnp.abs(ee - eo)
            avr = pltpu.roll(av, L - 1, 2)
            awr = pltpu.roll(aw, L - 1, 2)
            acc = acc + jnp.maximum(av, avr) + jnp.maximum(aw, awr)
    colsum = jnp.sum(acc.reshape(hh, L), axis=0, keepdims=True)
    lane = jax.lax.broadcasted_iota(jnp.int32, colsum.shape, 1)
    masked = jnp.where((lane & 1) == 0, colsum, 0.0)
    out_ref[...] = jnp.sum(masked, axis=1, keepdims=True)[None]


def kernel(pred, target):
    B, C, H, W = pred.shape
    S = B * C
    x5 = pred.reshape(S, H // 2, 2, W)
    y5 = target.reshape(S, H // 2, 2, W)

    N = S // _SL
    steps = N // 2

    any_spec = pl.BlockSpec(memory_space=pl.ANY)

    partials = pl.pallas_call(
        _dwt_l1_body,
        grid=(2, steps),
        in_specs=[any_spec, any_spec],
        out_specs=pl.BlockSpec(
            (1, 1, 1), lambda ci, i: (ci * steps + i, 0, 0)),
        out_shape=jax.ShapeDtypeStruct((N, 1, 1), jnp.float32),
        scratch_shapes=[
            pltpu.VMEM((_NBUF, _SL, H // 2, W), jnp.float32),
            pltpu.VMEM((_NBUF, _SL, H // 2, W), jnp.float32),
            pltpu.VMEM((_NBUF, _SL, H // 2, W), jnp.float32),
            pltpu.VMEM((_NBUF, _SL, H // 2, W), jnp.float32),
            pltpu.SemaphoreType.DMA((_NBUF,)),
            pltpu.SemaphoreType.DMA((_NBUF,)),
            pltpu.SemaphoreType.DMA((_NBUF,)),
            pltpu.SemaphoreType.DMA((_NBUF,)),
        ],
        compiler_params=pltpu.CompilerParams(
            dimension_semantics=("parallel", "arbitrary"),
            vmem_limit_bytes=62 * 1024 * 1024,
        ),
    )(x5, y5)

    n = S * (H // 2) * (W // 2)
    return jnp.sum(partials) * (1.0 / n)


# speedup vs baseline: 3.9457x; 3.9457x over previous
import jax
import jax.numpy as jnp
from jax.experimental import pallas as pl
from jax.experimental.pallas import tpu as pltpu


def _dwt_l1_body(p0_ref, p1_ref, t0_ref, t1_ref, out_ref):
    bb, cc, h, w = p0_ref.shape
    g = 32
    acc = jnp.zeros((g // 8, 8, w), jnp.float32)
    for p_ref, t_ref in ((p0_ref, t0_ref), (p1_ref, t1_ref)):
        for b in range(bb):
            for c in range(cc):
                for r0 in range(0, h, g):
                    e = (p_ref[b, c, r0:r0 + g] - t_ref[b, c, r0:r0 + g])
                    e = e.reshape(g // 8, 8, w)
                    e_dn = pltpu.roll(e, 7, 1)
                    av = jnp.abs(e + e_dn)
                    aw = jnp.abs(e - e_dn)
                    avr = jnp.concatenate([av[:, :, 1:], av[:, :, :1]], axis=2)
                    awr = jnp.concatenate([aw[:, :, 1:], aw[:, :, :1]], axis=2)
                    acc = acc + jnp.maximum(av, avr) + jnp.maximum(aw, awr)
    row = jax.lax.broadcasted_iota(jnp.int32, (1, 8, w), 1)
    acc = jnp.where((row & 1) == 0, acc, 0.0)
    colsum = jnp.sum(acc.reshape(g, w), axis=0, keepdims=True)
    lane = jax.lax.broadcasted_iota(jnp.int32, colsum.shape, 1)
    masked = jnp.where((lane & 1) == 0, colsum, 0.0)
    out_ref[...] = jnp.sum(masked, axis=1, keepdims=True)[None, None]


def kernel(pred, target):
    B, C, H, W = pred.shape

    BB = 2
    half0 = pl.BlockSpec((BB, C, H // 2, W), lambda i: (i, 0, 0, 0))
    half1 = pl.BlockSpec((BB, C, H // 2, W), lambda i: (i, 0, 1, 0))

    partials = pl.pallas_call(
        _dwt_l1_body,
        grid=(B // BB,),
        in_specs=[half0, half1, half0, half1],
        out_specs=pl.BlockSpec((1, 1, 1, 1), lambda i: (i, 0, 0, 0)),
        out_shape=jax.ShapeDtypeStruct((B // BB, 1, 1, 1), jnp.float32),
        compiler_params=pltpu.CompilerParams(
            dimension_semantics=("parallel",),
            vmem_limit_bytes=62 * 1024 * 1024,
        ),
    )(pred, pred, target, target)

    n = B * C * (H // 2) * (W // 2)
    return jnp.sum(partials) * (1.0 / n)


# speedup vs baseline: 3.9774x; 1.0080x over previous
import jax
import jax.numpy as jnp
from jax.experimental import pallas as pl
from jax.experimental.pallas import tpu as pltpu


def _dwt_l1_body(p0_ref, p1_ref, t0_ref, t1_ref, out_ref):
    bb, cc, h, w = p0_ref.shape
    g = 32
    L = 128
    acc = jnp.zeros((g // 8, 8, L), jnp.float32)
    for p_ref, t_ref in ((p0_ref, t0_ref), (p1_ref, t1_ref)):
        for b in range(bb):
            for c in range(cc):
                for r0 in range(0, h, g):
                    for c0 in range(0, w, L):
                        e = (p_ref[b, c, r0:r0 + g, c0:c0 + L]
                             - t_ref[b, c, r0:r0 + g, c0:c0 + L])
                        e = e.reshape(g // 8, 8, L)
                        e_dn = pltpu.roll(e, 7, 1)
                        av = jnp.abs(e + e_dn)
                        aw = jnp.abs(e - e_dn)
                        avr = pltpu.roll(av, L - 1, 2)
                        awr = pltpu.roll(aw, L - 1, 2)
                        acc = acc + jnp.maximum(av, avr) + jnp.maximum(aw, awr)
    row = jax.lax.broadcasted_iota(jnp.int32, (1, 8, L), 1)
    acc = jnp.where((row & 1) == 0, acc, 0.0)
    colsum = jnp.sum(acc.reshape(g, L), axis=0, keepdims=True)
    lane = jax.lax.broadcasted_iota(jnp.int32, colsum.shape, 1)
    masked = jnp.where((lane & 1) == 0, colsum, 0.0)
    out_ref[...] = jnp.sum(masked, axis=1, keepdims=True)[None, None]


def kernel(pred, target):
    B, C, H, W = pred.shape

    BB = 2
    half0 = pl.BlockSpec((BB, C, H // 2, W), lambda i: (i, 0, 0, 0))
    half1 = pl.BlockSpec((BB, C, H // 2, W), lambda i: (i, 0, 1, 0))

    partials = pl.pallas_call(
        _dwt_l1_body,
        grid=(B // BB,),
        in_specs=[half0, half1, half0, half1],
        out_specs=pl.BlockSpec((1, 1, 1, 1), lambda i: (i, 0, 0, 0)),
        out_shape=jax.ShapeDtypeStruct((B // BB, 1, 1, 1), jnp.float32),
        compiler_params=pltpu.CompilerParams(
            dimension_semantics=("parallel",),
            vmem_limit_bytes=62 * 1024 * 1024,
        ),
    )(pred, pred, target, target)

    n = B * C * (H // 2) * (W // 2)
    return jnp.sum(partials) * (1.0 / n)
